# baseline (speedup 1.0000x reference)
import functools

import jax
import jax.numpy as jnp
from jax import lax
from jax.experimental import pallas as pl
from jax.experimental.pallas import tpu as pltpu
from jax.experimental.pallas import tpu_sc as plsc

BATCH = 16384
DIM = 64
NC = 2
NS = 16
NW = NC * NS
BPW = BATCH // NW
CH = 128
NCH = BPW // CH
LANES = 16
GPC = CH // LANES

_mesh = plsc.VectorSubcoreMesh(core_axis_name="c", subcore_axis_name="s")


_STRIPS = 64
_BCOLS = _STRIPS * 128
_NBLK = -(-100000 // _BCOLS)
_OROWS = _STRIPS * DIM
_V2 = _NBLK * _OROWS


def _retile_body(wt_ref, ct_ref, ow_ref, oc_ref):
    eye = jnp.eye(DIM, dtype=jnp.float32)
    dn = (((0,), (0,)), ((), ()))
    yw = lax.dot_general(wt_ref[...], eye, dn,
                         preferred_element_type=jnp.float32)
    yc = lax.dot_general(ct_ref[...], eye, dn,
                         preferred_element_type=jnp.float32)
    for s in range(_STRIPS):
        r = 128 * s
        ow_ref[DIM * s:DIM * (s + 1), :] = jnp.concatenate(
            [yw[r:r + 64, :], yw[r + 64:r + 128, :]], axis=1)
        oc_ref[DIM * s:DIM * (s + 1), :] = jnp.concatenate(
            [yc[r:r + 64, :], yc[r + 64:r + 128, :]], axis=1)


def _retile(wt, ct):
    return pl.pallas_call(
        _retile_body,
        grid=(_NBLK,),
        in_specs=[
            pl.BlockSpec((DIM, _BCOLS), lambda j: (0, j)),
            pl.BlockSpec((DIM, _BCOLS), lambda j: (0, j)),
        ],
        out_specs=[
            pl.BlockSpec((_OROWS, 2 * DIM), lambda j: (j, 0)),
            pl.BlockSpec((_OROWS, 2 * DIM), lambda j: (j, 0)),
        ],
        out_shape=[
            jax.ShapeDtypeStruct((_V2, 2 * DIM), jnp.float32),
            jax.ShapeDtypeStruct((_V2, 2 * DIM), jnp.float32),
        ],
    )(wt, ct)


@functools.partial(
    pl.kernel,
    mesh=_mesh,
    compiler_params=pltpu.CompilerParams(needs_layout_passes=False),
    out_type=jax.ShapeDtypeStruct((BATCH,), jnp.float32),
    scratch_types=[
        pltpu.VMEM((NCH, CH), jnp.int32),
        pltpu.VMEM((NCH, CH), jnp.int32),
        pltpu.VMEM((NCH, CH), jnp.int32),
        pltpu.VMEM((NCH, CH), jnp.int32),
        pltpu.VMEM((CH, 2 * DIM), jnp.float32),
        pltpu.VMEM((CH, 2 * DIM), jnp.float32),
        pltpu.VMEM((CH, 2 * DIM), jnp.float32),
        pltpu.VMEM((CH, 2 * DIM), jnp.float32),
        pltpu.VMEM((BPW,), jnp.float32),
        pltpu.VMEM((BPW,), jnp.float32),
        pltpu.SemaphoreType.DMA,
        pltpu.SemaphoreType.DMA,
    ],
)
def _glove_sc(pw_hbm, pc_hbm, ow_hbm, oc_hbm, ww_hbm, wc_hbm, bsum_hbm,
              out_hbm, pw_v, pc_v, ow_v, oc_v,
              rows_w0, rows_w1, rows_c0, rows_c1, bias_v, out_v,
              sem0, sem1):
    wid = lax.axis_index("s") * NC + lax.axis_index("c")
    base = wid * BPW

    pltpu.sync_copy(pw_hbm.at[wid], pw_v)
    pltpu.sync_copy(pc_hbm.at[wid], pc_v)
    pltpu.sync_copy(ow_hbm.at[wid], ow_v)
    pltpu.sync_copy(oc_hbm.at[wid], oc_v)
    pltpu.sync_copy(bsum_hbm.at[pl.ds(base, BPW)], bias_v)

    rbufs = [(rows_w0, rows_c0, sem0), (rows_w1, rows_c1, sem1)]
    handles = [None] * NCH

    def fire(j):
        rw, rc, sem = rbufs[j % 2]
        handles[j] = [
            pltpu.async_copy(ww_hbm.at[pw_v.at[j]], rw, sem),
            pltpu.async_copy(wc_hbm.at[pc_v.at[j]], rc, sem),
        ]

    lane = lax.iota(jnp.int32, LANES)
    fzero = jnp.zeros((LANES,), jnp.float32)

    fire(0)
    for j in range(NCH):
        if j + 1 < NCH:
            fire(j + 1)
        rw, rc, _ = rbufs[j % 2]
        for h in handles[j]:
            h.wait()

        @plsc.parallel_loop(0, GPC, unroll=4)
        def group(g):
            rows = g * LANES + lane
            par_w = ow_v[j, pl.ds(g * LANES, LANES)]
            par_c = oc_v[j, pl.ds(g * LANES, LANES)]
            acc = [bias_v[pl.ds(j * CH + g * LANES, LANES)],
                   fzero, fzero, fzero]
            for d in range(DIM):
                acc[d % 4] = acc[d % 4] + (
                    plsc.load_gather(rw, [rows, par_w + d])
                    * plsc.load_gather(rc, [rows, par_c + d]))
            out_v[pl.ds(j * CH + g * LANES, LANES)] = (
                (acc[0] + acc[1]) + (acc[2] + acc[3]))

    pltpu.sync_copy(out_v, out_hbm.at[pl.ds(base, BPW)])


def kernel(word_idx, context_idx, W_word, W_ctx, b_word, b_ctx):
    wi = word_idx.astype(jnp.int32)
    ci = context_idx.astype(jnp.int32)
    pw = (((wi >> 7) << 6) + (wi & 63)).reshape(NW, NCH, CH)
    pc = (((ci >> 7) << 6) + (ci & 63)).reshape(NW, NCH, CH)
    ow = (((wi >> 6) & 1) << 6).reshape(NW, NCH, CH)
    oc = (((ci >> 6) & 1) << 6).reshape(NW, NCH, CH)
    w2w, w2c = _retile(W_word.T, W_ctx.T)
    bsum = (jnp.take(b_word, word_idx, axis=0)
            + jnp.take(b_ctx, context_idx, axis=0)).reshape(BATCH)
    out = _glove_sc(pw, pc, ow, oc, w2w, w2c, bsum)
    return out.reshape(BATCH, 1)


# speedup vs baseline: 1.0375x; 1.0375x over previous
import functools

import jax
import jax.numpy as jnp
from jax import lax
from jax.experimental import pallas as pl
from jax.experimental.pallas import tpu as pltpu
from jax.experimental.pallas import tpu_sc as plsc

BATCH = 16384
DIM = 64
NC = 2
NS = 16
NW = NC * NS
BPW = BATCH // NW
CH = 128
NCH = BPW // CH
LANES = 16
GPC = CH // LANES

_mesh = plsc.VectorSubcoreMesh(core_axis_name="c", subcore_axis_name="s")


_STRIPS = 64
_BCOLS = _STRIPS * 128
_NBLK = -(-100000 // _BCOLS)
_OROWS = _STRIPS * DIM
_V2 = _NBLK * _OROWS


def _retile_body(wt_ref, ct_ref, ow_ref, oc_ref):
    eye = jnp.eye(DIM, dtype=jnp.float32)
    dn = (((0,), (0,)), ((), ()))
    yw = lax.dot_general(wt_ref[...], eye, dn,
                         preferred_element_type=jnp.float32)
    yc = lax.dot_general(ct_ref[...], eye, dn,
                         preferred_element_type=jnp.float32)
    for s in range(_STRIPS):
        r = 128 * s
        ow_ref[DIM * s:DIM * (s + 1), :] = jnp.concatenate(
            [yw[r:r + 64, :], yw[r + 64:r + 128, :]], axis=1)
        oc_ref[DIM * s:DIM * (s + 1), :] = jnp.concatenate(
            [yc[r:r + 64, :], yc[r + 64:r + 128, :]], axis=1)


def _retile(wt, ct):
    return pl.pallas_call(
        _retile_body,
        grid=(_NBLK,),
        in_specs=[
            pl.BlockSpec((DIM, _BCOLS), lambda j: (0, j)),
            pl.BlockSpec((DIM, _BCOLS), lambda j: (0, j)),
        ],
        out_specs=[
            pl.BlockSpec((_OROWS, 2 * DIM), lambda j: (j, 0)),
            pl.BlockSpec((_OROWS, 2 * DIM), lambda j: (j, 0)),
        ],
        out_shape=[
            jax.ShapeDtypeStruct((_V2, 2 * DIM), jnp.float32),
            jax.ShapeDtypeStruct((_V2, 2 * DIM), jnp.float32),
        ],
    )(wt, ct)


@functools.partial(
    pl.kernel,
    mesh=_mesh,
    compiler_params=pltpu.CompilerParams(needs_layout_passes=False),
    out_type=jax.ShapeDtypeStruct((BATCH,), jnp.float32),
    scratch_types=[
        pltpu.VMEM((NCH, CH), jnp.int32),
        pltpu.VMEM((NCH, CH), jnp.int32),
        pltpu.VMEM((NCH, CH), jnp.int32),
        pltpu.VMEM((NCH, CH), jnp.int32),
        pltpu.VMEM((CH, 2 * DIM), jnp.float32),
        pltpu.VMEM((CH, 2 * DIM), jnp.float32),
        pltpu.VMEM((CH, 2 * DIM), jnp.float32),
        pltpu.VMEM((CH, 2 * DIM), jnp.float32),
        pltpu.VMEM((BPW,), jnp.float32),
        pltpu.VMEM((BPW,), jnp.float32),
        pltpu.SemaphoreType.DMA,
        pltpu.SemaphoreType.DMA,
    ],
)
def _glove_sc(pw_hbm, pc_hbm, ow_hbm, oc_hbm, ww_hbm, wc_hbm, bsum_hbm,
              out_hbm, pw_v, pc_v, ow_v, oc_v,
              rows_w0, rows_w1, rows_c0, rows_c1, bias_v, out_v,
              sem0, sem1):
    wid = lax.axis_index("s") * NC + lax.axis_index("c")
    base = wid * BPW

    pltpu.sync_copy(pw_hbm.at[wid], pw_v)
    pltpu.sync_copy(pc_hbm.at[wid], pc_v)
    pltpu.sync_copy(ow_hbm.at[wid], ow_v)
    pltpu.sync_copy(oc_hbm.at[wid], oc_v)
    pltpu.sync_copy(bsum_hbm.at[pl.ds(base, BPW)], bias_v)

    rbufs = [(rows_w0, rows_c0, sem0), (rows_w1, rows_c1, sem1)]
    handles = [None] * NCH

    def fire(j):
        rw, rc, sem = rbufs[j % 2]
        handles[j] = [
            pltpu.async_copy(ww_hbm.at[pw_v.at[j]], rw, sem),
            pltpu.async_copy(wc_hbm.at[pc_v.at[j]], rc, sem),
        ]

    lane = lax.iota(jnp.int32, LANES)
    fzero = jnp.zeros((LANES,), jnp.float32)

    fire(0)
    for j in range(NCH):
        if j + 1 < NCH:
            fire(j + 1)
        rw, rc, _ = rbufs[j % 2]
        for h in handles[j]:
            h.wait()

        @plsc.parallel_loop(0, GPC, unroll=2)
        def group(g):
            rows = g * LANES + lane
            par_w = ow_v[j, pl.ds(g * LANES, LANES)]
            par_c = oc_v[j, pl.ds(g * LANES, LANES)]
            acc = [bias_v[pl.ds(j * CH + g * LANES, LANES)],
                   fzero, fzero, fzero]
            for d in range(DIM):
                acc[d % 4] = acc[d % 4] + (
                    plsc.load_gather(rw, [rows, par_w + d])
                    * plsc.load_gather(rc, [rows, par_c + d]))
            out_v[pl.ds(j * CH + g * LANES, LANES)] = (
                (acc[0] + acc[1]) + (acc[2] + acc[3]))

    pltpu.sync_copy(out_v, out_hbm.at[pl.ds(base, BPW)])


def kernel(word_idx, context_idx, W_word, W_ctx, b_word, b_ctx):
    wi = word_idx.astype(jnp.int32)
    ci = context_idx.astype(jnp.int32)
    pw = (((wi >> 7) << 6) + (wi & 63)).reshape(NW, NCH, CH)
    pc = (((ci >> 7) << 6) + (ci & 63)).reshape(NW, NCH, CH)
    ow = (((wi >> 6) & 1) << 6).reshape(NW, NCH, CH)
    oc = (((ci >> 6) & 1) << 6).reshape(NW, NCH, CH)
    w2w, w2c = _retile(W_word.T, W_ctx.T)
    bsum = (jnp.take(b_word, word_idx, axis=0)
            + jnp.take(b_ctx, context_idx, axis=0)).reshape(BATCH)
    out = _glove_sc(pw, pc, ow, oc, w2w, w2c, bsum)
    return out.reshape(BATCH, 1)
